# baseline (speedup 1.0000x reference)
import jax
import jax.numpy as jnp
from jax.experimental import pallas as pl
from jax.experimental.pallas import tpu as pltpu

_BLOCK_ROWS = 31232


def _copy_block(src_ref, dst_ref):
    dst_ref[...] = src_ref[...]


def kernel(expressions_encodings, symbols_encodings,
           symbols_appearances_cfg_expression_idx,
           symbols_appearances_expression_token_idx,
           symbols_appearances_symbol_idx):
    orig_shape = expressions_encodings.shape
    b, t, d = orig_shape
    flat = expressions_encodings.reshape(b * t, d)
    n_rows = b * t
    block = min(_BLOCK_ROWS, n_rows)
    grid = (n_rows + block - 1) // block

    out = pl.pallas_call(
        _copy_block,
        grid=(grid,),
        in_specs=[pl.BlockSpec((block, d), lambda i: (i, 0))],
        out_specs=pl.BlockSpec((block, d), lambda i: (i, 0)),
        out_shape=jax.ShapeDtypeStruct((n_rows, d), flat.dtype),
        compiler_params=pltpu.CompilerParams(
            dimension_semantics=("arbitrary",),
            vmem_limit_bytes=100 * 1024 * 1024,
        ),
    )(flat)
    return out.reshape(orig_shape)
